# baseline (speedup 1.0000x reference)
import functools

import jax
import jax.numpy as jnp
from jax.experimental import pallas as pl
from jax.experimental.pallas import tpu as pltpu


def _mlp_kernel(feat_ref, cmt_ref, phi_ref, w1_ref, b1_ref, w2_ref,
                b2_ref, w3_ref, b3_ref, w4_ref, b4_ref, out_ref,
                *, bt, c, f, l):
    feat = feat_ref[0]
    cmt = cmt_ref[0]
    phi = phi_ref[0]

    w1f = w1_ref[0:f, :]
    w1p = w1_ref[2 * f:2 * f + l, :]

    dot = functools.partial(jnp.dot, preferred_element_type=jnp.float32)

    def silu(x):
        half = jnp.asarray(0.5, x.dtype)
        m = half * x
        return m + m * jnp.tanh(m)

    a = dot(feat, w1f) + (dot(phi, w1p) + b1_ref[...])
    b = jax.lax.dot_general(
        cmt, w1_ref[f:2 * f, :], (((0,), (0,)), ((), ())),
        preferred_element_type=jnp.float32)
    ab = a.astype(jnp.bfloat16)
    bb = b.astype(jnp.bfloat16)
    h1 = ab[:, None, :] + bb[None, :, :]
    h1 = silu(h1).reshape(bt * c, 512)

    h2 = silu(dot(h1, w2_ref[...]).astype(jnp.bfloat16) + b2_ref[...])
    h3 = silu(dot(h2, w3_ref[...]).astype(jnp.bfloat16) + b3_ref[...])

    e = dot(h3, w4_ref[...]).reshape(bt, c) + b4_ref[...]

    ones = jnp.ones((f, 1), jnp.bfloat16)
    f2 = dot(feat * feat, ones)
    c2 = jax.lax.dot_general(
        ones, cmt * cmt, (((0,), (0,)), ((), ())),
        preferred_element_type=jnp.float32)
    fc = dot(feat, cmt)
    base = jnp.sqrt(jnp.maximum(f2 + c2 - 2.0 * fc, 0.0))

    out_ref[...] = (e + base)[None]


def kernel(features, class_mean, phi, W1, b1, W2, b2, W3, b3, W4, b4):
    s, t, f = features.shape
    c = class_mean.shape[1]
    l = phi.shape[1]
    bt = 128
    nt = t // bt

    featb = features.astype(jnp.bfloat16)
    cmt = class_mean.transpose(0, 2, 1).astype(jnp.bfloat16)
    phi3 = phi.reshape(s, 1, l).astype(jnp.bfloat16)
    W1b = W1.astype(jnp.bfloat16)
    W2b = W2.astype(jnp.bfloat16)
    W3b = W3.astype(jnp.bfloat16)
    b1r = b1.reshape(1, -1)
    b2r = b2.reshape(1, -1).astype(jnp.bfloat16)
    b3r = b3.reshape(1, -1).astype(jnp.bfloat16)
    w4r = W4.astype(jnp.bfloat16)
    b4r = b4.reshape(1, 1)

    grid = (s, nt)
    out = pl.pallas_call(
        functools.partial(_mlp_kernel, bt=bt, c=c, f=f, l=l),
        grid=grid,
        in_specs=[
            pl.BlockSpec((1, bt, f), lambda i, j: (i, j, 0)),
            pl.BlockSpec((1, f, c), lambda i, j: (i, 0, 0)),
            pl.BlockSpec((1, 1, l), lambda i, j: (i, 0, 0)),
            pl.BlockSpec(W1b.shape, lambda i, j: (0, 0)),
            pl.BlockSpec(b1r.shape, lambda i, j: (0, 0)),
            pl.BlockSpec(W2b.shape, lambda i, j: (0, 0)),
            pl.BlockSpec(b2r.shape, lambda i, j: (0, 0)),
            pl.BlockSpec(W3b.shape, lambda i, j: (0, 0)),
            pl.BlockSpec(b3r.shape, lambda i, j: (0, 0)),
            pl.BlockSpec(w4r.shape, lambda i, j: (0, 0)),
            pl.BlockSpec(b4r.shape, lambda i, j: (0, 0)),
        ],
        out_specs=pl.BlockSpec((1, bt, c), lambda i, j: (i, j, 0)),
        out_shape=jax.ShapeDtypeStruct((s, t, c), jnp.float32),
        compiler_params=pltpu.CompilerParams(
            dimension_semantics=("parallel", "parallel"),
        ),
    )(featb, cmt, phi3, W1b, b1r, W2b, b2r, W3b, b3r, w4r, b4r)
    return out


# speedup vs baseline: 1.1382x; 1.1382x over previous
import functools

import jax
import jax.numpy as jnp
from jax.experimental import pallas as pl
from jax.experimental.pallas import tpu as pltpu


def _mlp_kernel(feat_ref, cm_ref, cmt_ref, phi_ref, w1_ref, b1_ref, w2_ref,
                b2_ref, w3_ref, b3_ref, w4_ref, b4_ref, out_ref,
                *, bt, c, f, l):
    feat = feat_ref[0]
    cm = cm_ref[0]
    cmt = cmt_ref[0]
    phi = phi_ref[0]

    w1f = w1_ref[0:f, :]
    w1c = w1_ref[f:2 * f, :]
    w1p = w1_ref[2 * f:2 * f + l, :]

    dot = functools.partial(jnp.dot, preferred_element_type=jnp.float32)

    def silu(x):
        half = jnp.asarray(0.5, x.dtype)
        m = half * x
        return m + m * jnp.tanh(m)

    a = dot(feat, w1f) + (dot(phi, w1p) + b1_ref[...])
    b = dot(cm, w1c)
    ab = a.astype(jnp.bfloat16)
    bb = b.astype(jnp.bfloat16)
    h1 = ab[:, None, :] + bb[None, :, :]
    h1 = silu(h1).reshape(bt * c, 512)

    h2 = silu(dot(h1, w2_ref[...]).astype(jnp.bfloat16) + b2_ref[...])
    h3 = silu(dot(h2, w3_ref[...]).astype(jnp.bfloat16) + b3_ref[...])

    e = dot(h3, w4_ref[...]).reshape(bt, c) + b4_ref[...]

    f2 = jnp.sum(feat * feat, axis=1, keepdims=True)
    c2 = jnp.sum(cmt * cmt, axis=0, keepdims=True)
    fc = dot(feat, cmt)
    base = jnp.sqrt(jnp.maximum(f2 + c2 - 2.0 * fc, 0.0))

    out_ref[...] = (e + base)[None]


def kernel(features, class_mean, phi, W1, b1, W2, b2, W3, b3, W4, b4):
    s, t, f = features.shape
    c = class_mean.shape[1]
    l = phi.shape[1]
    bt = 256
    nt = t // bt

    phi3 = phi.reshape(s, 1, l)
    cmt = class_mean.transpose(0, 2, 1)
    W2b = W2.astype(jnp.bfloat16)
    W3b = W3.astype(jnp.bfloat16)
    b1r = b1.reshape(1, -1)
    b2r = b2.reshape(1, -1).astype(jnp.bfloat16)
    b3r = b3.reshape(1, -1).astype(jnp.bfloat16)
    w4r = W4.astype(jnp.bfloat16)
    b4r = b4.reshape(1, 1)

    grid = (s, nt)
    out = pl.pallas_call(
        functools.partial(_mlp_kernel, bt=bt, c=c, f=f, l=l),
        grid=grid,
        in_specs=[
            pl.BlockSpec((1, bt, f), lambda i, j: (i, j, 0)),
            pl.BlockSpec((1, c, f), lambda i, j: (i, 0, 0)),
            pl.BlockSpec((1, f, c), lambda i, j: (i, 0, 0)),
            pl.BlockSpec((1, 1, l), lambda i, j: (i, 0, 0)),
            pl.BlockSpec(W1.shape, lambda i, j: (0, 0)),
            pl.BlockSpec(b1r.shape, lambda i, j: (0, 0)),
            pl.BlockSpec(W2b.shape, lambda i, j: (0, 0)),
            pl.BlockSpec(b2r.shape, lambda i, j: (0, 0)),
            pl.BlockSpec(W3b.shape, lambda i, j: (0, 0)),
            pl.BlockSpec(b3r.shape, lambda i, j: (0, 0)),
            pl.BlockSpec(w4r.shape, lambda i, j: (0, 0)),
            pl.BlockSpec(b4r.shape, lambda i, j: (0, 0)),
        ],
        out_specs=pl.BlockSpec((1, bt, c), lambda i, j: (i, j, 0)),
        out_shape=jax.ShapeDtypeStruct((s, t, c), jnp.float32),
        compiler_params=pltpu.CompilerParams(
            dimension_semantics=("parallel", "parallel"),
        ),
    )(features, class_mean, cmt, phi3, W1, b1r, W2b, b2r, W3b, b3r, w4r, b4r)
    return out


# speedup vs baseline: 1.1702x; 1.0281x over previous
import functools

import jax
import jax.numpy as jnp
from jax.experimental import pallas as pl
from jax.experimental.pallas import tpu as pltpu


def _mlp_kernel(feat_ref, cm_ref, cmt_ref, phi_ref, w1_ref, b1_ref, w2_ref,
                b2_ref, w3_ref, b3_ref, w4_ref, b4_ref, out_ref,
                *, bt, c, f, l):
    feat = feat_ref[0]
    cm = cm_ref[0]
    cmt = cmt_ref[0]
    phi = phi_ref[0]

    w1f = w1_ref[0:f, :]
    w1c = w1_ref[f:2 * f, :]
    w1p = w1_ref[2 * f:2 * f + l, :]

    dot = functools.partial(jnp.dot, preferred_element_type=jnp.float32)

    def silu(x):
        half = jnp.asarray(0.5, x.dtype)
        m = half * x
        return m + m * jnp.tanh(m)

    a = dot(feat, w1f) + (dot(phi, w1p) + b1_ref[...])
    b = dot(cm, w1c)
    ab = a.astype(jnp.bfloat16)
    bb = b.astype(jnp.bfloat16)
    h1 = ab[:, None, :] + bb[None, :, :]
    h1 = silu(h1).reshape(bt * c, 512)

    h2 = silu(dot(h1, w2_ref[...]).astype(jnp.bfloat16) + b2_ref[...])
    h3 = silu(dot(h2, w3_ref[...]).astype(jnp.bfloat16) + b3_ref[...])

    e = dot(h3, w4_ref[...]).reshape(bt, c) + b4_ref[...]

    f2 = jnp.sum(feat * feat, axis=1, keepdims=True)
    c2 = jnp.sum(cmt * cmt, axis=0, keepdims=True)
    fc = dot(feat, cmt)
    base = jnp.sqrt(jnp.maximum(f2 + c2 - 2.0 * fc, 0.0))

    out_ref[...] = (e + base)[None]


def kernel(features, class_mean, phi, W1, b1, W2, b2, W3, b3, W4, b4):
    s, t, f = features.shape
    c = class_mean.shape[1]
    l = phi.shape[1]
    bt = 512
    nt = t // bt

    phi3 = phi.reshape(s, 1, l)
    cmt = class_mean.transpose(0, 2, 1)
    W2b = W2.astype(jnp.bfloat16)
    W3b = W3.astype(jnp.bfloat16)
    b1r = b1.reshape(1, -1)
    b2r = b2.reshape(1, -1).astype(jnp.bfloat16)
    b3r = b3.reshape(1, -1).astype(jnp.bfloat16)
    w4r = W4.astype(jnp.bfloat16)
    b4r = b4.reshape(1, 1)

    grid = (s, nt)
    out = pl.pallas_call(
        functools.partial(_mlp_kernel, bt=bt, c=c, f=f, l=l),
        grid=grid,
        in_specs=[
            pl.BlockSpec((1, bt, f), lambda i, j: (i, j, 0)),
            pl.BlockSpec((1, c, f), lambda i, j: (i, 0, 0)),
            pl.BlockSpec((1, f, c), lambda i, j: (i, 0, 0)),
            pl.BlockSpec((1, 1, l), lambda i, j: (i, 0, 0)),
            pl.BlockSpec(W1.shape, lambda i, j: (0, 0)),
            pl.BlockSpec(b1r.shape, lambda i, j: (0, 0)),
            pl.BlockSpec(W2b.shape, lambda i, j: (0, 0)),
            pl.BlockSpec(b2r.shape, lambda i, j: (0, 0)),
            pl.BlockSpec(W3b.shape, lambda i, j: (0, 0)),
            pl.BlockSpec(b3r.shape, lambda i, j: (0, 0)),
            pl.BlockSpec(w4r.shape, lambda i, j: (0, 0)),
            pl.BlockSpec(b4r.shape, lambda i, j: (0, 0)),
        ],
        out_specs=pl.BlockSpec((1, bt, c), lambda i, j: (i, j, 0)),
        out_shape=jax.ShapeDtypeStruct((s, t, c), jnp.float32),
        compiler_params=pltpu.CompilerParams(
            dimension_semantics=("parallel", "parallel"),
        ),
    )(features, class_mean, cmt, phi3, W1, b1r, W2b, b2r, W3b, b3r, w4r, b4r)
    return out
